# baseline (speedup 1.0000x reference)
import functools

import jax
import jax.numpy as jnp
from jax import lax
from jax.experimental import pallas as pl
from jax.experimental.pallas import tpu as pltpu
from jax.experimental.pallas import tpu_sc as plsc

_NC = 2
_NS = 16
_DW = 16
_K = 40
_G = 5


def _mesh():
    return plsc.VectorSubcoreMesh(
        core_axis_name="c", subcore_axis_name="s",
        num_cores=_NC, num_subcores=_NS)


def _striped_copy(n, s, copy_fn):
    rpt8 = (n // _NS) // 8 * 8
    rem = n - _NS * rpt8
    copy_fn(pl.ds(s * rpt8, rpt8))
    if rem:
        @pl.when(s == _NS - 1)
        def _():
            copy_fn(pl.ds(_NS * rpt8, rem))


def _sc_degree(dst, init):
    e = dst.shape[0]
    n = init.shape[1]
    nw = _NC * _NS
    epw = e // nw
    gk = _G * _K
    ngrp = epw // gk
    npair = ngrp // 2

    @functools.partial(
        pl.kernel,
        out_type=jax.ShapeDtypeStruct((_NC, n, _DW), jnp.float32),
        mesh=_mesh(),
        scratch_types=[
            pltpu.VMEM((gk,), jnp.int32),
            pltpu.VMEM((gk,), jnp.int32),
            pltpu.VMEM((_K, _DW), jnp.float32),
            pltpu.VMEM_SHARED((n, _DW), jnp.float32),
        ] + [pltpu.SemaphoreType.DMA] * (_G + 2),
    )
    def deg_kernel(dst_hbm, init_hbm, out_hbm, dsta, dstb, ones, acc, *sems):
        semsc = sems[:_G]
        semia, semib = sems[_G:]
        c = lax.axis_index("c")
        s = lax.axis_index("s")
        base = (c * _NS + s) * epw

        def idx_slice(t):
            return dst_hbm.at[pl.ds(base + t * gk, gk)]

        def scatter_group(dref):
            scs = [pltpu.async_copy(
                       ones, acc.at[dref.at[pl.ds(i * _K, _K)]], semsc[i],
                       add=True)
                   for i in range(_G)]
            for d in scs:
                d.wait()

        @pl.loop(0, _K)
        def _(i):
            ones[i, :] = jnp.full((_DW,), 1.0, jnp.float32)

        _striped_copy(n, s, lambda sl: pltpu.sync_copy(
            init_hbm.at[c].at[sl], acc.at[sl]))
        pltpu.async_copy(idx_slice(0), dsta, semia)
        plsc.subcore_barrier()

        @pl.loop(0, npair - 1)
        def _(u):
            pltpu.make_async_copy(idx_slice(2 * u), dsta, semia).wait()
            pltpu.async_copy(idx_slice(2 * u + 1), dstb, semib)
            scatter_group(dsta)
            pltpu.make_async_copy(idx_slice(2 * u + 1), dstb, semib).wait()
            pltpu.async_copy(idx_slice(2 * u + 2), dsta, semia)
            scatter_group(dstb)

        pltpu.make_async_copy(idx_slice(ngrp - 2), dsta, semia).wait()
        pltpu.async_copy(idx_slice(ngrp - 1), dstb, semib)
        scatter_group(dsta)
        pltpu.make_async_copy(idx_slice(ngrp - 1), dstb, semib).wait()
        scatter_group(dstb)

        plsc.subcore_barrier()
        _striped_copy(n, s, lambda sl: pltpu.sync_copy(
            acc.at[sl], out_hbm.at[c].at[sl]))

    return deg_kernel(dst, init)


@functools.lru_cache(maxsize=None)
def _make_sc_aggregate(n, h, e):
    nw = _NC * _NS
    epw = e // nw
    kw = 80
    nwin = epw // kw
    assert nwin % 4 == 1 and nwin > 8

    @functools.partial(
        pl.kernel,
        out_type=jax.ShapeDtypeStruct((_NC, n, h), jnp.float32),
        mesh=_mesh(),
        scratch_types=[pltpu.VMEM((kw,), jnp.int32)] * 8
        + [pltpu.VMEM((kw, h), jnp.float32)] * 2
        + [pltpu.VMEM_SHARED((n, h), jnp.float32)]
        + [pltpu.SemaphoreType.DMA] * 8,
    )
    def agg_kernel(hs_hbm, src_hbm, dst_hbm, out_hbm, *rest):
        srcb = rest[0:4]
        dstb = rest[4:8]
        rows = rest[8:10]
        acc = rest[10]
        semg = rest[11:13]
        semsc = rest[13:15]
        semi = rest[15:19]
        c = lax.axis_index("c")
        s = lax.axis_index("s")
        base = (c * _NS + s) * epw

        def load_idx(t, m):
            sl = pl.ds(base + t * kw, kw)
            pltpu.async_copy(src_hbm.at[sl], srcb[m], semi[m])
            pltpu.async_copy(dst_hbm.at[sl], dstb[m], semi[m])

        def wait_idx(t, m):
            sl = pl.ds(base + t * kw, kw)
            pltpu.make_async_copy(src_hbm.at[sl], srcb[m], semi[m]).wait()
            pltpu.make_async_copy(dst_hbm.at[sl], dstb[m], semi[m]).wait()

        def fire_gather(m, p):
            pltpu.async_copy(hs_hbm.at[srcb[m]], rows[p], semg[p])

        def drain_gather(m, p):
            pltpu.make_async_copy(hs_hbm.at[srcb[m]], rows[p], semg[p]).wait()

        def fire_scatter(m, p):
            pltpu.async_copy(rows[p], acc.at[dstb[m]], semsc[p], add=True)

        def drain_scatter(m, p):
            pltpu.make_async_copy(rows[p], acc.at[dstb[m]], semsc[p]).wait()

        def phase(t, j, first=False, fire_next=True, load_next=True):
            m, p, q = j % 4, j % 2, 1 - j % 2
            drain_gather(m, p)
            fire_scatter(m, p)
            if not first:
                drain_scatter((j - 1) % 4, q)
            if fire_next:
                wait_idx(t + 1, (j + 1) % 4)
                fire_gather((j + 1) % 4, q)
            if load_next:
                load_idx(t + 3, (j + 3) % 4)

        _striped_copy(n, s, lambda sl: pltpu.sync_copy(
            hs_hbm.at[sl], acc.at[sl]))
        load_idx(0, 0)
        load_idx(1, 1)
        load_idx(2, 2)
        plsc.subcore_barrier()
        wait_idx(0, 0)
        fire_gather(0, 0)

        phase(0, 0, first=True)
        phase(1, 1)
        phase(2, 2)
        phase(3, 3)

        @pl.loop(1, (nwin - 5) // 4)
        def _(u):
            t = 4 * u
            phase(t, 0)
            phase(t + 1, 1)
            phase(t + 2, 2)
            phase(t + 3, 3)

        for t in range(nwin - 5, nwin):
            j = t % 4
            phase(t, j, fire_next=(t + 1 < nwin), load_next=(t + 3 < nwin))
        drain_scatter((nwin - 1) % 4, (nwin - 1) % 2)

        plsc.subcore_barrier()
        _striped_copy(n, s, lambda sl: pltpu.sync_copy(
            acc.at[sl], out_hbm.at[c].at[sl]))

    return jax.jit(agg_kernel)


def _sc_aggregate(hs, src, dst):
    n, h = hs.shape
    return _make_sc_aggregate(n, h, src.shape[0])(hs, src, dst)


def _tc_matmul(x, w):
    n = x.shape[0]
    h = w.shape[1]

    def body(x_ref, w_ref, o_ref):
        o_ref[...] = jnp.dot(x_ref[...], w_ref[...],
                             preferred_element_type=jnp.float32)

    return pl.pallas_call(
        body, out_shape=jax.ShapeDtypeStruct((n, h), jnp.float32))(x, w)


def _tc_scale(h1, degp):
    n, h = h1.shape

    def body(h_ref, deg_ref, hs_ref, dinv_ref):
        deg = deg_ref[0] + deg_ref[1]
        dinv = lax.rsqrt(deg)
        dcol = dinv[:, 0:1]
        hs_ref[...] = h_ref[...] * dcol
        dinv_ref[...] = dcol

    return pl.pallas_call(
        body,
        out_shape=[
            jax.ShapeDtypeStruct((n, h), jnp.float32),
            jax.ShapeDtypeStruct((n, 1), jnp.float32),
        ],
    )(h1, degp)


def _tc_mid(p, hs, dinv, b, g, be, w_next):
    n, h = hs.shape

    def body(p_ref, hs_ref, dinv_ref, b_ref, g_ref, be_ref, w_ref, o_ref):
        dcol = dinv_ref[...]
        z = dcol * (p_ref[0] + p_ref[1] - hs_ref[...]) + b_ref[...][None, :]
        mean = jnp.mean(z, axis=0, keepdims=True)
        zc = z - mean
        var = jnp.mean(zc * zc, axis=0, keepdims=True)
        zn = g_ref[...][None, :] * zc * lax.rsqrt(var + 1e-5) + be_ref[...][None, :]
        a = jnp.maximum(zn, 0.0)
        o_ref[...] = jnp.dot(a, w_ref[...], preferred_element_type=jnp.float32) * dcol

    return pl.pallas_call(
        body,
        out_shape=jax.ShapeDtypeStruct((n, h), jnp.float32),
    )(p, hs, dinv, b, g, be, w_next)


def _tc_fin(p, hs, dinv, b):
    n, h = hs.shape

    def body(p_ref, hs_ref, dinv_ref, b_ref, o_ref):
        dcol = dinv_ref[...]
        o_ref[...] = dcol * (p_ref[0] + p_ref[1] - hs_ref[...]) + b_ref[...][None, :]

    return pl.pallas_call(
        body,
        out_shape=jax.ShapeDtypeStruct((n, h), jnp.float32),
    )(p, hs, dinv, b)


def kernel(x, edge_index, W1, b1, W2, b2, W3, b3, g1, be1, g2, be2):
    n = x.shape[0]
    e = edge_index.shape[1]
    src = edge_index[0]
    dst = edge_index[1]
    init = jnp.concatenate(
        [jnp.ones((1, n, _DW), jnp.float32), jnp.zeros((1, n, _DW), jnp.float32)]
    )
    h1 = _tc_matmul(x, W1)
    degp = _sc_degree(dst, init)
    hs1, dinv = _tc_scale(h1, degp)
    p1 = _sc_aggregate(hs1, src, dst)
    hs2 = _tc_mid(p1, hs1, dinv, b1, g1, be1, W2)
    p2 = _sc_aggregate(hs2, src, dst)
    hs3 = _tc_mid(p2, hs2, dinv, b2, g2, be2, W3)
    p3 = _sc_aggregate(hs3, src, dst)
    return _tc_fin(p3, hs3, dinv, b3)


# speedup vs baseline: 1.1107x; 1.1107x over previous
import functools

import jax
import jax.numpy as jnp
from jax import lax
from jax.experimental import pallas as pl
from jax.experimental.pallas import tpu as pltpu
from jax.experimental.pallas import tpu_sc as plsc

_NC = 2
_NS = 16
_DW = 16
_K = 40
_G = 5


def _mesh():
    return plsc.VectorSubcoreMesh(
        core_axis_name="c", subcore_axis_name="s",
        num_cores=_NC, num_subcores=_NS)


def _striped_copy(n, s, copy_fn):
    rpt8 = (n // _NS) // 8 * 8
    rem = n - _NS * rpt8
    copy_fn(pl.ds(s * rpt8, rpt8))
    if rem:
        @pl.when(s == _NS - 1)
        def _():
            copy_fn(pl.ds(_NS * rpt8, rem))


def _sc_degree(dst, init):
    e = dst.shape[0]
    n = init.shape[1]
    nw = _NC * _NS
    epw = e // nw
    gk = _G * _K
    ngrp = epw // gk
    npair = ngrp // 2

    @functools.partial(
        pl.kernel,
        out_type=jax.ShapeDtypeStruct((_NC, n, _DW), jnp.float32),
        mesh=_mesh(),
        scratch_types=[
            pltpu.VMEM((gk,), jnp.int32),
            pltpu.VMEM((gk,), jnp.int32),
            pltpu.VMEM((_K, _DW), jnp.float32),
            pltpu.VMEM_SHARED((n, _DW), jnp.float32),
        ] + [pltpu.SemaphoreType.DMA] * (_G + 2),
    )
    def deg_kernel(dst_hbm, init_hbm, out_hbm, dsta, dstb, ones, acc, *sems):
        semsc = sems[:_G]
        semia, semib = sems[_G:]
        c = lax.axis_index("c")
        s = lax.axis_index("s")
        base = (c * _NS + s) * epw

        def idx_slice(t):
            return dst_hbm.at[pl.ds(base + t * gk, gk)]

        def scatter_group(dref):
            scs = [pltpu.async_copy(
                       ones, acc.at[dref.at[pl.ds(i * _K, _K)]], semsc[i],
                       add=True)
                   for i in range(_G)]
            for d in scs:
                d.wait()

        @pl.loop(0, _K)
        def _(i):
            ones[i, :] = jnp.full((_DW,), 1.0, jnp.float32)

        _striped_copy(n, s, lambda sl: pltpu.sync_copy(
            init_hbm.at[c].at[sl], acc.at[sl]))
        pltpu.async_copy(idx_slice(0), dsta, semia)
        plsc.subcore_barrier()

        @pl.loop(0, npair - 1)
        def _(u):
            pltpu.make_async_copy(idx_slice(2 * u), dsta, semia).wait()
            pltpu.async_copy(idx_slice(2 * u + 1), dstb, semib)
            scatter_group(dsta)
            pltpu.make_async_copy(idx_slice(2 * u + 1), dstb, semib).wait()
            pltpu.async_copy(idx_slice(2 * u + 2), dsta, semia)
            scatter_group(dstb)

        pltpu.make_async_copy(idx_slice(ngrp - 2), dsta, semia).wait()
        pltpu.async_copy(idx_slice(ngrp - 1), dstb, semib)
        scatter_group(dsta)
        pltpu.make_async_copy(idx_slice(ngrp - 1), dstb, semib).wait()
        scatter_group(dstb)

        plsc.subcore_barrier()
        _striped_copy(n, s, lambda sl: pltpu.sync_copy(
            acc.at[sl], out_hbm.at[c].at[sl]))

    return deg_kernel(dst, init)


@functools.lru_cache(maxsize=None)
def _make_sc_aggregate(n, h, e):
    nw = _NC * _NS
    epw = e // nw
    kw = 40
    nwin = epw // kw
    assert nwin > 9

    @functools.partial(
        pl.kernel,
        out_type=jax.ShapeDtypeStruct((_NC, n, h), jnp.float32),
        mesh=_mesh(),
        scratch_types=[pltpu.VMEM((kw,), jnp.int32)] * 8
        + [pltpu.VMEM((kw, h), jnp.float32)] * 4
        + [pltpu.VMEM_SHARED((n, h), jnp.float32)]
        + [pltpu.SemaphoreType.DMA] * 12,
    )
    def agg_kernel(hs_hbm, src_hbm, dst_hbm, out_hbm, *rest):
        srcb = rest[0:4]
        dstb = rest[4:8]
        rows = rest[8:12]
        acc = rest[12]
        semg = rest[13:17]
        semsc = rest[17:21]
        semi = rest[21:25]
        c = lax.axis_index("c")
        s = lax.axis_index("s")
        base = (c * _NS + s) * epw

        def load_idx(t, m):
            sl = pl.ds(base + t * kw, kw)
            pltpu.async_copy(src_hbm.at[sl], srcb[m], semi[m])
            pltpu.async_copy(dst_hbm.at[sl], dstb[m], semi[m])

        def wait_idx(t, m):
            sl = pl.ds(base + t * kw, kw)
            pltpu.make_async_copy(src_hbm.at[sl], srcb[m], semi[m]).wait()
            pltpu.make_async_copy(dst_hbm.at[sl], dstb[m], semi[m]).wait()

        def fire_gather(m):
            pltpu.async_copy(hs_hbm.at[srcb[m]], rows[m], semg[m])

        def drain_gather(m):
            pltpu.make_async_copy(hs_hbm.at[srcb[m]], rows[m], semg[m]).wait()

        def fire_scatter(m):
            pltpu.async_copy(rows[m], acc.at[dstb[m]], semsc[m], add=True)

        def drain_scatter(m):
            pltpu.make_async_copy(rows[m], acc.at[dstb[m]], semsc[m]).wait()

        def phase(t, j, first=False):
            drain_gather(j % 4)
            fire_scatter(j % 4)
            if not first:
                drain_scatter((j - 1) % 4)
            if not isinstance(t, int) or t + 2 < nwin:
                wait_idx(t + 2, (j + 2) % 4)
                fire_gather((j + 2) % 4)
            if not isinstance(t, int) or t + 3 < nwin:
                load_idx(t + 3, (j + 3) % 4)

        _striped_copy(n, s, lambda sl: pltpu.sync_copy(
            hs_hbm.at[sl], acc.at[sl]))
        load_idx(0, 0)
        load_idx(1, 1)
        load_idx(2, 2)
        plsc.subcore_barrier()
        wait_idx(0, 0)
        fire_gather(0)
        wait_idx(1, 1)
        fire_gather(1)

        phase(0, 0, first=True)
        phase(1, 1)
        phase(2, 2)
        phase(3, 3)
        nq = (nwin - 4) // 4

        @pl.loop(1, 1 + nq)
        def _(u):
            t = 4 * u
            phase(t, 0)
            phase(t + 1, 1)
            phase(t + 2, 2)
            phase(t + 3, 3)

        for t in range(4 + 4 * nq, nwin):
            phase(t, t % 4)
        drain_scatter((nwin - 1) % 4)

        plsc.subcore_barrier()
        _striped_copy(n, s, lambda sl: pltpu.sync_copy(
            acc.at[sl], out_hbm.at[c].at[sl]))

    return jax.jit(agg_kernel)


def _sc_aggregate(hs, src, dst):
    n, h = hs.shape
    return _make_sc_aggregate(n, h, src.shape[0])(hs, src, dst)


def _tc_matmul(x, w):
    n = x.shape[0]
    h = w.shape[1]

    def body(x_ref, w_ref, o_ref):
        o_ref[...] = jnp.dot(x_ref[...], w_ref[...],
                             preferred_element_type=jnp.float32)

    return pl.pallas_call(
        body, out_shape=jax.ShapeDtypeStruct((n, h), jnp.float32))(x, w)


def _tc_scale(h1, degp):
    n, h = h1.shape

    def body(h_ref, deg_ref, hs_ref, dinv_ref):
        deg = deg_ref[0] + deg_ref[1]
        dinv = lax.rsqrt(deg)
        dcol = dinv[:, 0:1]
        hs_ref[...] = h_ref[...] * dcol
        dinv_ref[...] = dcol

    return pl.pallas_call(
        body,
        out_shape=[
            jax.ShapeDtypeStruct((n, h), jnp.float32),
            jax.ShapeDtypeStruct((n, 1), jnp.float32),
        ],
    )(h1, degp)


def _tc_mid(p, hs, dinv, b, g, be, w_next):
    n, h = hs.shape

    def body(p_ref, hs_ref, dinv_ref, b_ref, g_ref, be_ref, w_ref, o_ref):
        dcol = dinv_ref[...]
        z = dcol * (p_ref[0] + p_ref[1] - hs_ref[...]) + b_ref[...][None, :]
        mean = jnp.mean(z, axis=0, keepdims=True)
        zc = z - mean
        var = jnp.mean(zc * zc, axis=0, keepdims=True)
        zn = g_ref[...][None, :] * zc * lax.rsqrt(var + 1e-5) + be_ref[...][None, :]
        a = jnp.maximum(zn, 0.0)
        o_ref[...] = jnp.dot(a, w_ref[...], preferred_element_type=jnp.float32) * dcol

    return pl.pallas_call(
        body,
        out_shape=jax.ShapeDtypeStruct((n, h), jnp.float32),
    )(p, hs, dinv, b, g, be, w_next)


def _tc_fin(p, hs, dinv, b):
    n, h = hs.shape

    def body(p_ref, hs_ref, dinv_ref, b_ref, o_ref):
        dcol = dinv_ref[...]
        o_ref[...] = dcol * (p_ref[0] + p_ref[1] - hs_ref[...]) + b_ref[...][None, :]

    return pl.pallas_call(
        body,
        out_shape=jax.ShapeDtypeStruct((n, h), jnp.float32),
    )(p, hs, dinv, b)


def kernel(x, edge_index, W1, b1, W2, b2, W3, b3, g1, be1, g2, be2):
    n = x.shape[0]
    e = edge_index.shape[1]
    src = edge_index[0]
    dst = edge_index[1]
    init = jnp.concatenate(
        [jnp.ones((1, n, _DW), jnp.float32), jnp.zeros((1, n, _DW), jnp.float32)]
    )
    h1 = _tc_matmul(x, W1)
    degp = _sc_degree(dst, init)
    hs1, dinv = _tc_scale(h1, degp)
    p1 = _sc_aggregate(hs1, src, dst)
    hs2 = _tc_mid(p1, hs1, dinv, b1, g1, be1, W2)
    p2 = _sc_aggregate(hs2, src, dst)
    hs3 = _tc_mid(p2, hs2, dinv, b2, g2, be2, W3)
    p3 = _sc_aggregate(hs3, src, dst)
    return _tc_fin(p3, hs3, dinv, b3)


# speedup vs baseline: 1.3833x; 1.2454x over previous
import functools

import jax
import jax.numpy as jnp
from jax import lax
from jax.experimental import pallas as pl
from jax.experimental.pallas import tpu as pltpu
from jax.experimental.pallas import tpu_sc as plsc

_NC = 2
_NS = 16
_DW = 16
_K = 40
_G = 5


def _mesh():
    return plsc.VectorSubcoreMesh(
        core_axis_name="c", subcore_axis_name="s",
        num_cores=_NC, num_subcores=_NS)


def _striped_copy(n, s, copy_fn):
    rpt8 = (n // _NS) // 8 * 8
    rem = n - _NS * rpt8
    copy_fn(pl.ds(s * rpt8, rpt8))
    if rem:
        @pl.when(s == _NS - 1)
        def _():
            copy_fn(pl.ds(_NS * rpt8, rem))


def _sc_degree(dst, init):
    e = dst.shape[0]
    n = init.shape[1]
    nw = _NC * _NS
    epw = e // nw
    gk = _G * _K
    ngrp = epw // gk
    npair = ngrp // 2

    @functools.partial(
        pl.kernel,
        out_type=jax.ShapeDtypeStruct((_NC, n, _DW), jnp.float32),
        mesh=_mesh(),
        scratch_types=[
            pltpu.VMEM((gk,), jnp.int32),
            pltpu.VMEM((gk,), jnp.int32),
            pltpu.VMEM((_K, _DW), jnp.float32),
            pltpu.VMEM_SHARED((n, _DW), jnp.float32),
        ] + [pltpu.SemaphoreType.DMA] * (_G + 2),
    )
    def deg_kernel(dst_hbm, init_hbm, out_hbm, dsta, dstb, ones, acc, *sems):
        semsc = sems[:_G]
        semia, semib = sems[_G:]
        c = lax.axis_index("c")
        s = lax.axis_index("s")
        base = (c * _NS + s) * epw

        def idx_slice(t):
            return dst_hbm.at[pl.ds(base + t * gk, gk)]

        def scatter_group(dref):
            scs = [pltpu.async_copy(
                       ones, acc.at[dref.at[pl.ds(i * _K, _K)]], semsc[i],
                       add=True)
                   for i in range(_G)]
            for d in scs:
                d.wait()

        @pl.loop(0, _K)
        def _(i):
            ones[i, :] = jnp.full((_DW,), 1.0, jnp.float32)

        _striped_copy(n, s, lambda sl: pltpu.sync_copy(
            init_hbm.at[c].at[sl], acc.at[sl]))
        pltpu.async_copy(idx_slice(0), dsta, semia)
        plsc.subcore_barrier()

        @pl.loop(0, npair - 1)
        def _(u):
            pltpu.make_async_copy(idx_slice(2 * u), dsta, semia).wait()
            pltpu.async_copy(idx_slice(2 * u + 1), dstb, semib)
            scatter_group(dsta)
            pltpu.make_async_copy(idx_slice(2 * u + 1), dstb, semib).wait()
            pltpu.async_copy(idx_slice(2 * u + 2), dsta, semia)
            scatter_group(dstb)

        pltpu.make_async_copy(idx_slice(ngrp - 2), dsta, semia).wait()
        pltpu.async_copy(idx_slice(ngrp - 1), dstb, semib)
        scatter_group(dsta)
        pltpu.make_async_copy(idx_slice(ngrp - 1), dstb, semib).wait()
        scatter_group(dstb)

        plsc.subcore_barrier()
        _striped_copy(n, s, lambda sl: pltpu.sync_copy(
            acc.at[sl], out_hbm.at[c].at[sl]))

    return deg_kernel(dst, init)


@functools.lru_cache(maxsize=None)
def _make_sc_aggregate(n, h, e):
    nw = _NC * _NS
    epw = e // nw
    kw = 80
    nwin = epw // kw
    assert nwin > 9

    @functools.partial(
        pl.kernel,
        out_type=jax.ShapeDtypeStruct((_NC, n, h), jnp.float32),
        mesh=_mesh(),
        scratch_types=[pltpu.VMEM((kw,), jnp.int32)] * 8
        + [pltpu.VMEM((kw, h), jnp.float32)] * 4
        + [pltpu.VMEM_SHARED((n, h), jnp.float32)]
        + [pltpu.SemaphoreType.DMA] * 12,
    )
    def agg_kernel(hs_hbm, src_hbm, dst_hbm, out_hbm, *rest):
        srcb = rest[0:4]
        dstb = rest[4:8]
        rows = rest[8:12]
        acc = rest[12]
        semg = rest[13:17]
        semsc = rest[17:21]
        semi = rest[21:25]
        c = lax.axis_index("c")
        s = lax.axis_index("s")
        base = (c * _NS + s) * epw

        def load_idx(t, m):
            sl = pl.ds(base + t * kw, kw)
            pltpu.async_copy(src_hbm.at[sl], srcb[m], semi[m])
            pltpu.async_copy(dst_hbm.at[sl], dstb[m], semi[m])

        def wait_idx(t, m):
            sl = pl.ds(base + t * kw, kw)
            pltpu.make_async_copy(src_hbm.at[sl], srcb[m], semi[m]).wait()
            pltpu.make_async_copy(dst_hbm.at[sl], dstb[m], semi[m]).wait()

        def fire_gather(m):
            pltpu.async_copy(hs_hbm.at[srcb[m]], rows[m], semg[m])

        def drain_gather(m):
            pltpu.make_async_copy(hs_hbm.at[srcb[m]], rows[m], semg[m]).wait()

        def fire_scatter(m):
            pltpu.async_copy(rows[m], acc.at[dstb[m]], semsc[m], add=True)

        def drain_scatter(m):
            pltpu.make_async_copy(rows[m], acc.at[dstb[m]], semsc[m]).wait()

        def phase(t, j, first=False):
            drain_gather(j % 4)
            fire_scatter(j % 4)
            if not first:
                drain_scatter((j - 1) % 4)
            if not isinstance(t, int) or t + 2 < nwin:
                wait_idx(t + 2, (j + 2) % 4)
                fire_gather((j + 2) % 4)
            if not isinstance(t, int) or t + 3 < nwin:
                load_idx(t + 3, (j + 3) % 4)

        _striped_copy(n, s, lambda sl: pltpu.sync_copy(
            hs_hbm.at[sl], acc.at[sl]))
        load_idx(0, 0)
        load_idx(1, 1)
        load_idx(2, 2)
        plsc.subcore_barrier()
        wait_idx(0, 0)
        fire_gather(0)
        wait_idx(1, 1)
        fire_gather(1)

        phase(0, 0, first=True)
        phase(1, 1)
        phase(2, 2)
        phase(3, 3)
        nq = (nwin - 4) // 4

        @pl.loop(1, 1 + nq)
        def _(u):
            t = 4 * u
            phase(t, 0)
            phase(t + 1, 1)
            phase(t + 2, 2)
            phase(t + 3, 3)

        for t in range(4 + 4 * nq, nwin):
            phase(t, t % 4)
        drain_scatter((nwin - 1) % 4)

        plsc.subcore_barrier()
        _striped_copy(n, s, lambda sl: pltpu.sync_copy(
            acc.at[sl], out_hbm.at[c].at[sl]))

    return jax.jit(agg_kernel)


def _sc_aggregate(hs, src, dst):
    n, h = hs.shape
    return _make_sc_aggregate(n, h, src.shape[0])(hs, src, dst)


def _tc_matmul(x, w):
    n = x.shape[0]
    h = w.shape[1]

    def body(x_ref, w_ref, o_ref):
        o_ref[...] = jnp.dot(x_ref[...], w_ref[...],
                             preferred_element_type=jnp.float32)

    return pl.pallas_call(
        body, out_shape=jax.ShapeDtypeStruct((n, h), jnp.float32))(x, w)


def _tc_scale(h1, degp):
    n, h = h1.shape

    def body(h_ref, deg_ref, hs_ref, dinv_ref):
        deg = deg_ref[0] + deg_ref[1]
        dinv = lax.rsqrt(deg)
        dcol = dinv[:, 0:1]
        hs_ref[...] = h_ref[...] * dcol
        dinv_ref[...] = dcol

    return pl.pallas_call(
        body,
        out_shape=[
            jax.ShapeDtypeStruct((n, h), jnp.float32),
            jax.ShapeDtypeStruct((n, 1), jnp.float32),
        ],
    )(h1, degp)


def _tc_mid(p, hs, dinv, b, g, be, w_next):
    n, h = hs.shape

    def body(p_ref, hs_ref, dinv_ref, b_ref, g_ref, be_ref, w_ref, o_ref):
        dcol = dinv_ref[...]
        z = dcol * (p_ref[0] + p_ref[1] - hs_ref[...]) + b_ref[...][None, :]
        mean = jnp.mean(z, axis=0, keepdims=True)
        zc = z - mean
        var = jnp.mean(zc * zc, axis=0, keepdims=True)
        zn = g_ref[...][None, :] * zc * lax.rsqrt(var + 1e-5) + be_ref[...][None, :]
        a = jnp.maximum(zn, 0.0)
        o_ref[...] = jnp.dot(a, w_ref[...], preferred_element_type=jnp.float32) * dcol

    return pl.pallas_call(
        body,
        out_shape=jax.ShapeDtypeStruct((n, h), jnp.float32),
    )(p, hs, dinv, b, g, be, w_next)


def _tc_fin(p, hs, dinv, b):
    n, h = hs.shape

    def body(p_ref, hs_ref, dinv_ref, b_ref, o_ref):
        dcol = dinv_ref[...]
        o_ref[...] = dcol * (p_ref[0] + p_ref[1] - hs_ref[...]) + b_ref[...][None, :]

    return pl.pallas_call(
        body,
        out_shape=jax.ShapeDtypeStruct((n, h), jnp.float32),
    )(p, hs, dinv, b)


def kernel(x, edge_index, W1, b1, W2, b2, W3, b3, g1, be1, g2, be2):
    n = x.shape[0]
    e = edge_index.shape[1]
    src = edge_index[0]
    dst = edge_index[1]
    init = jnp.concatenate(
        [jnp.ones((1, n, _DW), jnp.float32), jnp.zeros((1, n, _DW), jnp.float32)]
    )
    h1 = _tc_matmul(x, W1)
    degp = _sc_degree(dst, init)
    hs1, dinv = _tc_scale(h1, degp)
    p1 = _sc_aggregate(hs1, src, dst)
    hs2 = _tc_mid(p1, hs1, dinv, b1, g1, be1, W2)
    p2 = _sc_aggregate(hs2, src, dst)
    hs3 = _tc_mid(p2, hs2, dinv, b2, g2, be2, W3)
    p3 = _sc_aggregate(hs3, src, dst)
    return _tc_fin(p3, hs3, dinv, b3)
